# Initial kernel scaffold; baseline (speedup 1.0000x reference)
import jax
import jax.numpy as jnp
from jax.experimental import pallas as pl


def kernel(hidden_states, gate_w, gate_b, wi_w, wi_b, wo_w, wo_b, lora_A, lora_B):
    raise NotImplementedError("write your pallas kernel here")


# speedup vs baseline: 6.0715x; 6.0715x over previous
import functools

import jax
import jax.numpy as jnp
from jax.experimental import pallas as pl

D_MODEL = 1024
D_FF = 4096
E = 8
RANK = 4
ER = E * RANK
TM = 512


def _moe_tile(x_ref, gate_w_ref, gate_b_ref, a_all_ref, b_cat_ref,
              wi_w_ref, wi_b_ref, wo_w_ref, wo_b_ref, out_ref):
    x = x_ref[...]

    logits = jax.lax.dot_general(
        x, gate_w_ref[...], (((1,), (1,)), ((), ())),
        preferred_element_type=jnp.float32) + gate_b_ref[...]
    m = jnp.max(logits, axis=-1, keepdims=True)
    idx = jax.lax.broadcasted_iota(jnp.int32, logits.shape, 1)
    e_sel = jnp.min(jnp.where(logits >= m, idx, E), axis=-1, keepdims=True)

    a = jax.lax.dot_general(
        x, a_all_ref[...], (((1,), (1,)), ((), ())),
        preferred_element_type=jnp.float32)
    col_expert = jax.lax.broadcasted_iota(jnp.int32, a.shape, 1) // RANK
    a_masked = jnp.where(col_expert == e_sel, a, 0.0)

    base = jax.lax.dot_general(
        x, wi_w_ref[...], (((1,), (1,)), ((), ())),
        preferred_element_type=jnp.float32)
    lora = jax.lax.dot_general(
        a_masked, b_cat_ref[...], (((1,), (1,)), ((), ())),
        preferred_element_type=jnp.float32)
    inter = jnp.maximum(base + lora + wi_b_ref[...], 0.0)

    out_ref[...] = jax.lax.dot_general(
        inter, wo_w_ref[...], (((1,), (1,)), ((), ())),
        preferred_element_type=jnp.float32) + wo_b_ref[...]


@functools.partial(jax.jit, static_argnames=("interpret",))
def _moe_forward(x, gate_w, gate_b, a_all, b_cat, wi_w, wi_b, wo_w, wo_b,
                 interpret=False):
    t = x.shape[0]
    grid = (t // TM,)
    full = lambda shape: pl.BlockSpec(shape, lambda i: (0,) * len(shape))
    return pl.pallas_call(
        _moe_tile,
        grid=grid,
        in_specs=[
            pl.BlockSpec((TM, D_MODEL), lambda i: (i, 0)),
            full((E, D_MODEL)),
            full((1, E)),
            full((ER, D_MODEL)),
            full((D_FF, ER)),
            full((D_FF, D_MODEL)),
            full((1, D_FF)),
            full((D_MODEL, D_FF)),
            full((1, D_MODEL)),
        ],
        out_specs=pl.BlockSpec((TM, D_MODEL), lambda i: (i, 0)),
        out_shape=jax.ShapeDtypeStruct((t, D_MODEL), jnp.float32),
        interpret=interpret,
    )(x, gate_w, gate_b, a_all, b_cat, wi_w, wi_b, wo_w, wo_b)


def kernel(hidden_states, gate_w, gate_b, wi_w, wi_b, wo_w, wo_b,
           lora_A, lora_B, interpret=False):
    b, s, d = hidden_states.shape
    x = hidden_states.reshape(b * s, d)
    a_all = lora_A.reshape(ER, D_MODEL)
    b_cat = jnp.transpose(lora_B, (1, 0, 2)).reshape(D_FF, ER)
    out = _moe_forward(x, gate_w, gate_b.reshape(1, E), a_all, b_cat,
                       wi_w, wi_b.reshape(1, D_FF), wo_w,
                       wo_b.reshape(1, D_MODEL), interpret=interpret)
    return out.reshape(b, s, d)
